# baseline (speedup 1.0000x reference)
import functools

import jax
import jax.numpy as jnp
from jax import lax
from jax.experimental import pallas as pl
from jax.experimental.pallas import tpu as pltpu
from jax.experimental.pallas import tpu_sc as plsc

_B = 16384
_D = 64
_N_GATHER = 6
_CHUNK = 128


@functools.lru_cache(maxsize=None)
def _build():
    info = plsc.get_sparse_core_info()
    nc, ns = info.num_cores, info.num_subcores
    nw = nc * ns
    bpw = _B // nw
    nj = bpw // _CHUNK
    mesh = plsc.VectorSubcoreMesh(core_axis_name="c", subcore_axis_name="s")
    out_type = tuple(
        jax.ShapeDtypeStruct((_B, _D), jnp.float32) for _ in range(_N_GATHER)
    )

    @functools.partial(
        pl.kernel,
        mesh=mesh,
        out_type=out_type,
        compiler_params=pltpu.CompilerParams(use_tc_tiling_on_sc=False),
        scratch_types=[
            pltpu.VMEM((_N_GATHER, nj, _CHUNK), jnp.int32),
            pltpu.VMEM((bpw, _D), jnp.float32),
            pltpu.SemaphoreType.DMA,
        ],
    )
    def gather6(users_hbm, tracks_hbm, i_u, i_tp, i_tn, i_up, i_un, i_ta,
                o_u, o_tp, o_tn, o_up, o_un, o_ta, idx_v, rows_v, sem):
        wid = lax.axis_index("s") * nc + lax.axis_index("c")
        base = wid * bpw
        tables = (users_hbm, tracks_hbm, tracks_hbm,
                  users_hbm, users_hbm, tracks_hbm)
        idxs = (i_u, i_tp, i_tn, i_up, i_un, i_ta)
        outs = (o_u, o_tp, o_tn, o_up, o_un, o_ta)
        for g in range(_N_GATHER):
            pltpu.sync_copy(idxs[g].at[pl.ds(wid * nj, nj)], idx_v.at[g])
            for j in range(nj):
                pltpu.async_copy(
                    tables[g].at[idx_v.at[g, j]],
                    rows_v.at[pl.ds(j * _CHUNK, _CHUNK)],
                    sem,
                ).wait()
            pltpu.sync_copy(rows_v, outs[g].at[pl.ds(base, bpw)])

    return gather6


def kernel(x_user, x_track_pos, x_track_neg, x_user_pos, x_user_neg,
           x_track_anchor, users_vecs, tracks_vecs):
    gather6 = _build()
    idx2d = [
        x.reshape(_B // _CHUNK, _CHUNK)
        for x in (x_user, x_track_pos, x_track_neg, x_user_pos, x_user_neg,
                  x_track_anchor)
    ]
    u, tp, tn, up, un, ta = gather6(users_vecs, tracks_vecs, *idx2d)
    return (u, tp, tn, up, un, ta)


# speedup vs baseline: 1.0648x; 1.0648x over previous
import functools

import jax
import jax.numpy as jnp
from jax import lax
from jax.experimental import pallas as pl
from jax.experimental.pallas import tpu as pltpu
from jax.experimental.pallas import tpu_sc as plsc

_B = 16384
_D = 64
_N_GATHER = 6
_CHUNK = 128
_NBUF = 4


@functools.lru_cache(maxsize=None)
def _build():
    info = plsc.get_sparse_core_info()
    nc, ns = info.num_cores, info.num_subcores
    nw = nc * ns
    bpw = _B // nw
    nj = bpw // _CHUNK
    nt = _N_GATHER * nj
    mesh = plsc.VectorSubcoreMesh(core_axis_name="c", subcore_axis_name="s")
    out_type = tuple(
        jax.ShapeDtypeStruct((_B, _D), jnp.float32) for _ in range(_N_GATHER)
    )

    @functools.partial(
        pl.kernel,
        mesh=mesh,
        out_type=out_type,
        compiler_params=pltpu.CompilerParams(use_tc_tiling_on_sc=False),
        scratch_types=[
            pltpu.VMEM((_N_GATHER, nj, _CHUNK), jnp.int32),
            pltpu.VMEM((_NBUF, _CHUNK, _D), jnp.float32),
            pltpu.SemaphoreType.DMA,
        ]
        + [pltpu.SemaphoreType.DMA] * _NBUF
        + [pltpu.SemaphoreType.DMA] * _NBUF,
    )
    def gather6(users_hbm, tracks_hbm, i_u, i_tp, i_tn, i_up, i_un, i_ta,
                o_u, o_tp, o_tn, o_up, o_un, o_ta, idx_v, rows_v, sem_i,
                *sems):
        sem_g = sems[:_NBUF]
        sem_s = sems[_NBUF:]
        wid = lax.axis_index("s") * nc + lax.axis_index("c")
        base = wid * bpw
        tables = (users_hbm, tracks_hbm, tracks_hbm,
                  users_hbm, users_hbm, tracks_hbm)
        idxs = (i_u, i_tp, i_tn, i_up, i_un, i_ta)
        outs = (o_u, o_tp, o_tn, o_up, o_un, o_ta)
        steps = [(g, j) for g in range(_N_GATHER) for j in range(nj)]

        icopies = [
            pltpu.async_copy(idxs[g].at[pl.ds(wid * nj, nj)], idx_v.at[g],
                             sem_i)
            for g in range(_N_GATHER)
        ]
        for c in icopies:
            c.wait()

        gcopies = [None] * nt
        scopies = [None] * nt
        for t in range(nt + 1):
            if t < nt:
                g, j = steps[t]
                b = t % _NBUF
                if t >= _NBUF:
                    scopies[t - _NBUF].wait()
                gcopies[t] = pltpu.async_copy(
                    tables[g].at[idx_v.at[g, j]], rows_v.at[b], sem_g[b])
            if t >= 1:
                g, j = steps[t - 1]
                b = (t - 1) % _NBUF
                gcopies[t - 1].wait()
                scopies[t - 1] = pltpu.async_copy(
                    rows_v.at[b],
                    outs[g].at[pl.ds(base + j * _CHUNK, _CHUNK)],
                    sem_s[b])
        for t in range(nt - _NBUF, nt):
            scopies[t].wait()

    return gather6


def kernel(x_user, x_track_pos, x_track_neg, x_user_pos, x_user_neg,
           x_track_anchor, users_vecs, tracks_vecs):
    gather6 = _build()
    idx2d = [
        x.reshape(_B // _CHUNK, _CHUNK)
        for x in (x_user, x_track_pos, x_track_neg, x_user_pos, x_user_neg,
                  x_track_anchor)
    ]
    u, tp, tn, up, un, ta = gather6(users_vecs, tracks_vecs, *idx2d)
    return (u, tp, tn, up, un, ta)
